# baseline (speedup 1.0000x reference)
import jax
import jax.numpy as jnp
from jax.experimental import pallas as pl

N_NODES = 10000
NUM_GRAPHS = 16
BN_EPS = 1e-5


def _gcn(x, src, dst, W, b):
    loop = jnp.arange(x.shape[0], dtype=src.dtype)
    s = jnp.concatenate([src, loop])
    d = jnp.concatenate([dst, loop])
    deg = jnp.zeros((x.shape[0],), dtype=x.dtype).at[d].add(1.0)
    dinv = jnp.where(deg > 0, deg ** -0.5, 0.0)
    norm = dinv[s] * dinv[d]
    h = x @ W
    msg = h[s] * norm[:, None]
    out = jnp.zeros_like(h).at[d].add(msg)
    return out + b


def _head_body(o_ref, mean_ref, var_ref, gam_ref, bet_ref, out_ref):
    o = o_ref[...]
    o = (o - mean_ref[...]) / jnp.sqrt(var_ref[...] + BN_EPS) * gam_ref[...] + bet_ref[...]
    out_ref[...] = jax.nn.sigmoid(o)


def kernel(x, edge_index, batch, W1, b1, W2, b2, fc1_W, fc1_b, fc2_W, fc2_b,
           bn_gamma, bn_beta, bn_mean, bn_var):
    h = x
    h = jax.nn.relu(_gcn(h, edge_index[0], edge_index[1], W1, b1))
    h = jax.nn.relu(_gcn(h, edge_index[0], edge_index[1], W2, b2))
    pooled = jax.ops.segment_max(h, batch, num_segments=NUM_GRAPHS)
    o = jax.nn.relu(pooled @ fc1_W + fc1_b)
    o = o @ fc2_W + fc2_b
    return pl.pallas_call(
        _head_body,
        out_shape=jax.ShapeDtypeStruct(o.shape, o.dtype),
    )(o, bn_mean[None, :], bn_var[None, :], bn_gamma[None, :], bn_beta[None, :])


# speedup vs baseline: 2.6436x; 2.6436x over previous
import functools

import jax
import jax.numpy as jnp
from jax import lax
from jax.experimental import pallas as pl
from jax.experimental.pallas import tpu as pltpu
from jax.experimental.pallas import tpu_sc as plsc

N_NODES = 10000
N_EDGES = 320000
NUM_GRAPHS = 16
BN_EPS = 1e-5

NP = 10240
NC, NS = 2, 16
NT = NC * NS
CHUNK = 128
NCH = 80
EP = NT * NCH * CHUNK
RB = 1024
NRB = NP // RB

_mesh = plsc.VectorSubcoreMesh(core_axis_name="c", subcore_axis_name="s")


def _make_gather_kernel(width):
    def body(tab_hbm, src_hbm, out_hbm, srcbuf, rows, sem):
        c = lax.axis_index("c")
        s = lax.axis_index("s")
        w = c * NS + s
        pltpu.sync_copy(src_hbm.at[w], srcbuf)

        def step(j, carry):
            pltpu.async_copy(tab_hbm.at[srcbuf.at[j]], rows, sem).wait()
            pltpu.sync_copy(rows, out_hbm.at[pl.ds((w * NCH + j) * CHUNK, CHUNK)])
            return carry

        lax.fori_loop(0, NCH, step, 0)

    return functools.partial(
        pl.kernel,
        out_type=jax.ShapeDtypeStruct((EP, width), jnp.float32),
        mesh=_mesh,
        scratch_types=[
            pltpu.VMEM((NCH, CHUNK), jnp.int32),
            pltpu.VMEM((CHUNK, width), jnp.float32),
            pltpu.SemaphoreType.DMA,
        ],
    )(body)


_gather256 = _make_gather_kernel(256)
_gather512 = _make_gather_kernel(512)


def _b2_body(deg_ref, x_ref, W1_ref, dinv_ref, g1_ref):
    dinv = lax.rsqrt(deg_ref[...] + 1.0)
    dinv_ref[...] = dinv
    h1 = jnp.dot(x_ref[...], W1_ref[...], preferred_element_type=jnp.float32)
    g1_ref[...] = h1 * dinv[:, 0:1]


def _b2_kernel(deg2d, xp, W1):
    return pl.pallas_call(
        _b2_body,
        grid=(NRB,),
        in_specs=[
            pl.BlockSpec((RB, 128), lambda i: (i, 0)),
            pl.BlockSpec((RB, 128), lambda i: (i, 0)),
            pl.BlockSpec((128, 256), lambda i: (0, 0)),
        ],
        out_specs=[
            pl.BlockSpec((RB, 128), lambda i: (i, 0)),
            pl.BlockSpec((RB, 256), lambda i: (i, 0)),
        ],
        out_shape=[
            jax.ShapeDtypeStruct((NP, 128), jnp.float32),
            jax.ShapeDtypeStruct((NP, 256), jnp.float32),
        ],
    )(deg2d, xp, W1)


def _d_body(acc_ref, g1_ref, dinv_ref, b1_ref, W2_ref, g2_ref):
    dinv1 = dinv_ref[...][:, 0:1]
    a1 = (acc_ref[...] + g1_ref[...]) * dinv1 + b1_ref[...]
    a1 = jnp.maximum(a1, 0.0)
    h2 = jnp.dot(a1, W2_ref[...], preferred_element_type=jnp.float32)
    g2_ref[...] = h2 * dinv1


def _d_kernel(acc1, g1, dinv, b1, W2):
    return pl.pallas_call(
        _d_body,
        grid=(NRB,),
        in_specs=[
            pl.BlockSpec((RB, 256), lambda i: (i, 0)),
            pl.BlockSpec((RB, 256), lambda i: (i, 0)),
            pl.BlockSpec((RB, 128), lambda i: (i, 0)),
            pl.BlockSpec((1, 256), lambda i: (0, 0)),
            pl.BlockSpec((256, 512), lambda i: (0, 0)),
        ],
        out_specs=pl.BlockSpec((RB, 512), lambda i: (i, 0)),
        out_shape=jax.ShapeDtypeStruct((NP, 512), jnp.float32),
    )(acc1, g1, dinv, b1, W2)


def _f_body(acc_ref, g2_ref, dinv_ref, b2_ref, batch_ref, out_ref, pooled_scr):
    i = pl.program_id(0)

    @pl.when(i == 0)
    def _():
        pooled_scr[...] = jnp.full((NUM_GRAPHS, 512), -jnp.inf, jnp.float32)

    dinv1 = dinv_ref[...][:, 0:1]
    h = (acc_ref[...] + g2_ref[...]) * dinv1 + b2_ref[...]
    h = jnp.maximum(h, 0.0)
    b = batch_ref[...]
    for g in range(NUM_GRAPHS):
        vals = jnp.where(b == g, h, -jnp.inf)
        mx = jnp.max(vals, axis=0, keepdims=True)
        pooled_scr[g:g + 1, :] = jnp.maximum(pooled_scr[g:g + 1, :], mx)

    @pl.when(i == NRB - 1)
    def _():
        out_ref[...] = pooled_scr[...]


def _f_kernel(acc2, g2, dinv, b2, batch_p):
    return pl.pallas_call(
        _f_body,
        grid=(NRB,),
        in_specs=[
            pl.BlockSpec((RB, 512), lambda i: (i, 0)),
            pl.BlockSpec((RB, 512), lambda i: (i, 0)),
            pl.BlockSpec((RB, 128), lambda i: (i, 0)),
            pl.BlockSpec((1, 512), lambda i: (0, 0)),
            pl.BlockSpec((RB, 1), lambda i: (i, 0)),
        ],
        out_specs=pl.BlockSpec((NUM_GRAPHS, 512), lambda i: (0, 0)),
        out_shape=jax.ShapeDtypeStruct((NUM_GRAPHS, 512), jnp.float32),
        scratch_shapes=[pltpu.VMEM((NUM_GRAPHS, 512), jnp.float32)],
    )(acc2, g2, dinv, b2, batch_p)


def _g_body(p_ref, w1_ref, bb1_ref, w2_ref, bb2_ref, gam_ref, bet_ref,
            mean_ref, var_ref, out_ref):
    o = jnp.dot(p_ref[...], w1_ref[...], preferred_element_type=jnp.float32)
    o = jnp.maximum(o + bb1_ref[...], 0.0)
    o = jnp.dot(o, w2_ref[...], preferred_element_type=jnp.float32) + bb2_ref[...]
    o = (o - mean_ref[...]) * lax.rsqrt(var_ref[...] + BN_EPS) * gam_ref[...] + bet_ref[...]
    out_ref[...] = jax.nn.sigmoid(o)


def _g_kernel(pooled, fc1_W, fc1_b, fc2_W, fc2_b, gam, bet, mean, var):
    return pl.pallas_call(
        _g_body,
        out_shape=jax.ShapeDtypeStruct((NUM_GRAPHS, 256), jnp.float32),
    )(pooled, fc1_W, fc1_b[None, :], fc2_W, fc2_b[None, :],
      gam[None, :], bet[None, :], mean[None, :], var[None, :])


def kernel(x, edge_index, batch, W1, b1, W2, b2, fc1_W, fc1_b, fc2_W, fc2_b,
           bn_gamma, bn_beta, bn_mean, bn_var):
    pad = EP - N_EDGES
    src = jnp.concatenate([edge_index[0], jnp.full((pad,), N_NODES, jnp.int32)])
    dst = jnp.concatenate([edge_index[1], jnp.full((pad,), N_NODES, jnp.int32)])
    src_ch = src.reshape(NT, NCH, CHUNK)
    xp = jnp.concatenate([x, jnp.zeros((NP - N_NODES, 128), jnp.float32)])
    batch_p = jnp.concatenate(
        [batch, jnp.full((NP - N_NODES,), NUM_GRAPHS, jnp.int32)]).reshape(NP, 1)

    deg = jnp.zeros((NP,), jnp.float32).at[dst].add(1.0)
    deg2d = jnp.broadcast_to(deg[:, None], (NP, 128))

    dinv, g1 = _b2_kernel(deg2d, xp, W1)
    msg1 = _gather256(g1, src_ch)
    acc1 = jnp.zeros((NP, 256), jnp.float32).at[dst].add(msg1)
    g2 = _d_kernel(acc1, g1, dinv, b1[None, :], W2)
    msg2 = _gather512(g2, src_ch)
    acc2 = jnp.zeros((NP, 512), jnp.float32).at[dst].add(msg2)
    pooled = _f_kernel(acc2, g2, dinv, b2[None, :], batch_p)
    return _g_kernel(pooled, fc1_W, fc1_b, fc2_W, fc2_b,
                     bn_gamma, bn_beta, bn_mean, bn_var)
